# Initial kernel scaffold; baseline (speedup 1.0000x reference)
import jax
import jax.numpy as jnp
from jax.experimental import pallas as pl


def kernel(x, phase_offsets):
    raise NotImplementedError("write your pallas kernel here")


# speedup vs baseline: 7.7199x; 7.7199x over previous
import functools

import jax
import jax.numpy as jnp
from jax import lax
from jax.experimental import pallas as pl
from jax.experimental.pallas import tpu as pltpu
from jax.experimental.pallas import tpu_sc as plsc

_B, _C, _L = 64, 64, 16384
_R = _B * _C
_PAD = 16
_NW = 32
_RPW = _R // _NW


def _body(x_hbm, ph_hbm, out_hbm,
          in0, in1, ot0, ot1, phv,
          isem0, isem1, osem0, osem1):
  ncores = 2
  wid = lax.axis_index("s") * ncores + lax.axis_index("c")
  base = wid * _RPW

  pltpu.sync_copy(ph_hbm.at[pl.ds(base, _RPW)], phv)

  iota = lax.broadcasted_iota(jnp.int32, (16,), 0)
  slots = ((in0, ot0, isem0, osem0), (in1, ot1, isem1, osem1))

  pltpu.async_copy(x_hbm.at[base], in0.at[pl.ds(_PAD, _L)], isem0)
  pltpu.async_copy(x_hbm.at[base + 1], in1.at[pl.ds(_PAD, _L)], isem1)

  @pl.loop(0, _RPW // 2)
  def _row_pair(g):
    for b, (inb, outb, isem, osem) in enumerate(slots):
      r = g * 2 + b
      pltpu.make_async_copy(x_hbm.at[base], inb.at[pl.ds(_PAD, _L)],
                            isem).wait()

      pvec = plsc.load_gather(phv, [jnp.broadcast_to(r, (16,))])
      idx0 = pvec + iota + _PAD

      @pl.when(r >= 2)
      def _():
        pltpu.make_async_copy(outb, out_hbm.at[base], osem).wait()

      @plsc.parallel_loop(16, _L - 16, 16, unroll=8)
      def _chunk(i):
        outb[pl.ds(i, 16)] = plsc.load_gather(inb, [idx0 + i])

      lpos = iota + pvec
      outb[pl.ds(0, 16)] = plsc.load_gather(
          inb, [jnp.where(lpos < 0, -lpos, lpos) + _PAD])
      tpos = (_L - 16) + iota + pvec
      outb[pl.ds(_L - 16, 16)] = plsc.load_gather(
          inb, [jnp.where(tpos >= _L, 2 * (_L - 1) - tpos, tpos) + _PAD])

      pltpu.async_copy(outb, out_hbm.at[base + r], osem)

      @pl.when(r + 2 < _RPW)
      def _():
        pltpu.async_copy(x_hbm.at[base + r + 2], inb.at[pl.ds(_PAD, _L)],
                         isem)

  pltpu.make_async_copy(ot0, out_hbm.at[base], osem0).wait()
  pltpu.make_async_copy(ot1, out_hbm.at[base], osem1).wait()


@jax.jit
def kernel(x, phase_offsets):
  xr = x.reshape(_R, _L)
  ph = phase_offsets.reshape(_R).astype(jnp.int32)

  mesh = plsc.VectorSubcoreMesh(core_axis_name="c", subcore_axis_name="s")
  run = pl.kernel(
      _body,
      out_type=jax.ShapeDtypeStruct((_R, _L), jnp.float32),
      mesh=mesh,
      compiler_params=pltpu.CompilerParams(needs_layout_passes=False,
                                           use_tc_tiling_on_sc=False),
      scratch_types=[
          pltpu.VMEM((_L + 2 * _PAD,), jnp.float32),
          pltpu.VMEM((_L + 2 * _PAD,), jnp.float32),
          pltpu.VMEM((_L,), jnp.float32),
          pltpu.VMEM((_L,), jnp.float32),
          pltpu.VMEM((_RPW,), jnp.int32),
          pltpu.SemaphoreType.DMA,
          pltpu.SemaphoreType.DMA,
          pltpu.SemaphoreType.DMA,
          pltpu.SemaphoreType.DMA,
      ],
  )
  out = run(xr, ph)
  return out.reshape(_B, _C, _L)


# speedup vs baseline: 16.9489x; 2.1955x over previous
import jax
import jax.numpy as jnp
from jax import lax
from jax.experimental import pallas as pl
from jax.experimental.pallas import tpu as pltpu
from jax.experimental.pallas import tpu_sc as plsc

_B, _C, _L = 64, 64, 16384
_R = _B * _C
_NW = 32
_GPW = 16
_NK = 8
_CW = _L // _NK
_SW = _CW + 256
_SRC = [min(max(k * _CW - 128, 0), _L - _SW) for k in range(_NK)]


def _body(x_hbm, ph_hbm, out_hbm, in0, in1, ot0, ot1, phv,
          isem0, isem1, osem0, osem1):
  ncores = 2
  wid = lax.axis_index("s") * ncores + lax.axis_index("c")
  base_row = wid * _GPW * 8

  pltpu.sync_copy(ph_hbm.at[pl.ds(base_row, _GPW * 8)], phv)

  iota = lax.broadcasted_iota(jnp.int32, (16,), 0)
  ins = (in0, in1)
  outs = (ot0, ot1)
  isems = (isem0, isem1)
  osems = (osem0, osem1)

  def in_start(gi, k, s):
    row0 = base_row + gi * 8
    pltpu.async_copy(x_hbm.at[pl.ds(row0, 8), pl.ds(_SRC[k], _SW)],
                     ins[s], isems[s])

  def in_wait(s):
    pltpu.make_async_copy(x_hbm.at[pl.ds(0, 8), pl.ds(0, _SW)],
                          ins[s], isems[s]).wait()

  def out_start(gi, k, s):
    row0 = base_row + gi * 8
    pltpu.async_copy(outs[s], out_hbm.at[pl.ds(row0, 8),
                                         pl.ds(k * _CW, _CW)], osems[s])

  def out_wait(s):
    pltpu.make_async_copy(outs[s], out_hbm.at[pl.ds(0, 8), pl.ds(0, _CW)],
                          osems[s]).wait()

  in_start(0, 0, 0)
  in_start(0, 1, 1)

  @pl.loop(0, _GPW)
  def _group(gi):
    pvs = [plsc.load_gather(phv, [jnp.broadcast_to(gi * 8 + rr, (16,))])
           for rr in range(8)]

    for k in range(_NK):
      s = k % 2
      it = gi * _NK + k
      in_wait(s)

      @pl.when(it >= 2)
      def _():
        out_wait(s)

      inb, outb = ins[s], outs[s]
      for rr in range(8):
        rrv = jnp.broadcast_to(rr, (16,))
        q0 = pvs[rr] + iota + (k * _CW - _SRC[k])

        lo = 16 if k == 0 else 0
        hi = _CW - 16 if k == _NK - 1 else _CW

        @plsc.parallel_loop(lo, hi, 16, unroll=8)
        def _chunk(c):
          plsc.store_scatter(outb, [rrv, iota + c],
                             plsc.load_gather(inb, [rrv, q0 + c]))

        if k == 0:
          q = pvs[rr] + iota
          qr = jnp.where(q < 0, -q, q)
          plsc.store_scatter(outb, [rrv, iota],
                             plsc.load_gather(inb, [rrv, qr]))
        if k == _NK - 1:
          q = pvs[rr] + iota + (_L - 16)
          qr = jnp.where(q >= _L, 2 * (_L - 1) - q, q)
          plsc.store_scatter(outb, [rrv, iota + (_CW - 16)],
                             plsc.load_gather(inb, [rrv, qr - _SRC[k]]))

      out_start(gi, k, s)

      @pl.when(it + 2 < _GPW * _NK)
      def _():
        gi_next = gi + 1 if k >= _NK - 2 else gi
        in_start(gi_next, (k + 2) % _NK, s)

  out_wait(0)
  out_wait(1)


@jax.jit
def kernel(x, phase_offsets):
  xr = x.reshape(_R, _L)
  ph = phase_offsets.reshape(_R).astype(jnp.int32)

  mesh = plsc.VectorSubcoreMesh(core_axis_name="c", subcore_axis_name="s")
  run = pl.kernel(
      _body,
      out_type=jax.ShapeDtypeStruct((_R, _L), jnp.float32),
      mesh=mesh,
      compiler_params=pltpu.CompilerParams(needs_layout_passes=False,
                                           use_tc_tiling_on_sc=True),
      scratch_types=[
          pltpu.VMEM((8, _SW), jnp.float32),
          pltpu.VMEM((8, _SW), jnp.float32),
          pltpu.VMEM((8, _CW), jnp.float32),
          pltpu.VMEM((8, _CW), jnp.float32),
          pltpu.VMEM((_GPW * 8,), jnp.int32),
          pltpu.SemaphoreType.DMA,
          pltpu.SemaphoreType.DMA,
          pltpu.SemaphoreType.DMA,
          pltpu.SemaphoreType.DMA,
      ],
  )
  out = run(xr, ph)
  return out.reshape(_B, _C, _L)
